# baseline (speedup 1.0000x reference)
import functools

import jax
import jax.numpy as jnp
from jax import lax
from jax.experimental import pallas as pl
from jax.experimental.pallas import tpu as pltpu
from jax.experimental.pallas import tpu_sc as plsc

_TEMP = 0.2
_THRES = 0.4
_NPI = 256
_D = 512
_NB = 32
_NLVL = 3


def _stage1_body(iou_ref, crop_ref, box_ref, out_ref):
    x = box_ref[...]
    z = crop_ref[0]
    nb2 = jnp.sum(x * x, axis=1, keepdims=True)
    inv_nb = jax.lax.rsqrt(jnp.maximum(nb2, 1e-24))
    nz2 = jnp.sum(z * z, axis=1, keepdims=True)
    inv_nz = jax.lax.rsqrt(jnp.maximum(nz2, 1e-24))
    zh = z * inv_nz
    dots = jax.lax.dot_general(
        x, zh, (((1,), (1,)), ((), ())),
        preferred_element_type=jnp.float32)
    cos = dots * inv_nb
    mask = (iou_ref[0] >= _THRES).astype(jnp.float32)
    sp = jnp.sum(cos * mask, axis=0, keepdims=True)
    sa = jnp.sum(cos, axis=0, keepdims=True)
    cp = jnp.sum(mask)
    lane = jax.lax.broadcasted_iota(jnp.int32, (1, 128), 1)
    row = jnp.zeros((1, 128), jnp.float32)
    for k in range(_NLVL):
        row = jnp.where(lane == 16 * k, sp[0, k], row)
        row = jnp.where(lane == 16 * (k + 3), sa[0, k], row)
    row = jnp.where(lane == 16 * 6, cp, row)
    out_ref[0] = row


def _stage2_body(stats_ref, binv_ref, out_ref):
    s = stats_ref[:, 0, :]
    cp = s[:, 96:97]
    cn = _NPI - cp
    lvl_tot = None
    for k in range(_NLVL):
        sp = s[:, 16 * k:16 * k + 1]
        sa = s[:, 16 * (k + 3):16 * (k + 3) + 1]
        sn = sa - sp
        sim_pos = -(sp / cp)
        sim_neg = -(sn / cn)
        pos = jnp.exp(sim_pos / _TEMP)
        neg = jnp.exp(sim_neg / _TEMP)
        lb = -jnp.log(pos / (pos + neg))
        lvl = jnp.sum(lb, axis=0, keepdims=True)
        lvl_tot = lvl if lvl_tot is None else jnp.minimum(lvl_tot, lvl)
    out_ref[...] = lvl_tot * binv_ref[0, 0]


def _stage1_tc(box, crop, iou3):
    return pl.pallas_call(
        _stage1_body,
        grid=(_NB,),
        in_specs=[
            pl.BlockSpec((1, _NPI, 1), lambda b: (b, 0, 0)),
            pl.BlockSpec((1, _NLVL, _D), lambda b: (b, 0, 0)),
            pl.BlockSpec((_NPI, _D), lambda b: (b, 0)),
        ],
        out_specs=pl.BlockSpec((1, 1, 128), lambda b: (b, 0, 0)),
        out_shape=jax.ShapeDtypeStruct((_NB, 1, 128), jnp.float32),
    )(iou3, crop, box)


def _stage2(stats, binv):
    return pl.pallas_call(
        _stage2_body,
        in_specs=[
            pl.BlockSpec((_NB, 1, 128), lambda: (0, 0, 0)),
            pl.BlockSpec(memory_space=pltpu.SMEM),
        ],
        out_specs=pl.BlockSpec((1, 1), lambda: (0, 0)),
        out_shape=jax.ShapeDtypeStruct((1, 1), jnp.float32),
    )(stats, binv)


_HALF = 128
_RUNROLL = 8
_NCH = _D // 16


def _rsqrt16(x):
    i = lax.bitcast_convert_type(x, jnp.int32)
    i = 0x5F3759DF - lax.shift_right_logical(i, 1)
    y = lax.bitcast_convert_type(i, jnp.float32)
    for _ in range(3):
        y = y * (1.5 - 0.5 * x * y * y)
    return y


def _bsum16(v):
    return jnp.full((16,), jnp.sum(v), jnp.float32)


_BLK = 64
_NBLK = _NPI // _BLK


def _sc_stage1_body(box_hbm, crop_hbm, out_hbm,
                    xb0, xb1, ob0, ob1, zbuf,
                    sem_a, sem_b, sem_oa, sem_ob):
    wid = lax.axis_index("s") * 2 + lax.axis_index("c")
    base_row = wid * _NPI
    pltpu.sync_copy(crop_hbm.at[wid], zbuf)
    xbufs = [xb0, xb1]
    obufs = [ob0, ob1]
    sems_in = [sem_a, sem_b]
    sems_out = [sem_oa, sem_ob]
    zero = jnp.zeros((16,), jnp.float32)

    in_cp = [None, None]
    out_cp = [None, None]
    in_cp[0] = pltpu.async_copy(
        box_hbm.at[pl.ds(base_row, _BLK)], xb0, sem_a)
    for blk in range(_NBLK):
        par = blk % 2
        in_cp[par].wait()
        if blk + 1 < _NBLK:
            nxt = (blk + 1) % 2
            in_cp[nxt] = pltpu.async_copy(
                box_hbm.at[pl.ds(base_row + (blk + 1) * _BLK, _BLK)],
                xbufs[nxt], sems_in[nxt])
        if out_cp[par] is not None:
            out_cp[par].wait()
        xbuf = xbufs[par]
        obuf = obufs[par]

        def group_body(g, carry, xbuf=xbuf, obuf=obuf):
            r0 = g * _RUNROLL
            accs = [[zero, zero, zero, zero] for _ in range(_RUNROLL)]
            for c in range(_NCH):
                z0 = zbuf[0, pl.ds(16 * c, 16)]
                z1 = zbuf[1, pl.ds(16 * c, 16)]
                z2 = zbuf[2, pl.ds(16 * c, 16)]
                for j in range(_RUNROLL):
                    x = xbuf[r0 + j, pl.ds(16 * c, 16)]
                    a = accs[j]
                    a[0] = a[0] + x * x
                    a[1] = a[1] + x * z0
                    a[2] = a[2] + x * z1
                    a[3] = a[3] + x * z2
            for j in range(_RUNROLL):
                for k in range(4):
                    obuf[r0 + j, pl.ds(16 * k, 16)] = accs[j][k]
            return carry

        lax.fori_loop(0, _BLK // _RUNROLL, group_body, 0)
        out_cp[par] = pltpu.async_copy(
            obuf, out_hbm.at[pl.ds(base_row + blk * _BLK, _BLK)],
            sems_out[par])
    out_cp[0].wait()
    out_cp[1].wait()


def _stage1_sc(box, cropT):
    mesh = plsc.VectorSubcoreMesh(core_axis_name="c", subcore_axis_name="s")
    f = functools.partial(
        pl.kernel,
        out_type=jax.ShapeDtypeStruct((_NB * _NPI, 64), jnp.float32),
        mesh=mesh,
        compiler_params=pltpu.CompilerParams(needs_layout_passes=False),
        scratch_types=[
            pltpu.VMEM((_BLK, _D), jnp.float32),
            pltpu.VMEM((_BLK, _D), jnp.float32),
            pltpu.VMEM((_BLK, 64), jnp.float32),
            pltpu.VMEM((_BLK, 64), jnp.float32),
            pltpu.VMEM((_NLVL, _D), jnp.float32),
            pltpu.SemaphoreType.DMA,
            pltpu.SemaphoreType.DMA,
            pltpu.SemaphoreType.DMA,
            pltpu.SemaphoreType.DMA,
        ],
    )(_sc_stage1_body)
    return f(box, cropT)


def _tc_post_body(iou_ref, crop_ref, parts_ref, binv_ref, out_ref, acc_ref):
    b = pl.program_id(0)
    p = parts_ref[...]
    nb2 = jnp.sum(p[:, 0:16], axis=1, keepdims=True)
    inv_nb = jax.lax.rsqrt(jnp.maximum(nb2, 1e-24))
    z = crop_ref[0]
    mask = (iou_ref[0] >= _THRES).astype(jnp.float32)
    cp = jnp.sum(mask, axis=0, keepdims=True)
    cn = _NPI - cp
    lane = jax.lax.broadcasted_iota(jnp.int32, (1, 128), 1)
    contrib = jnp.zeros((1, 128), jnp.float32)
    for li in range(_NLVL):
        d = jnp.sum(p[:, 16 * (li + 1):16 * (li + 2)], axis=1, keepdims=True)
        zi = z[li:li + 1, :]
        nz2 = jnp.sum(zi * zi)
        cos = d * inv_nb * jax.lax.rsqrt(jnp.maximum(nz2, 1e-24))
        sp = jnp.sum(cos * mask, axis=0, keepdims=True)
        sa = jnp.sum(cos, axis=0, keepdims=True)
        sim_pos = -(sp / cp)
        sim_neg = -((sa - sp) / cn)
        pos = jnp.exp(sim_pos / _TEMP)
        neg = jnp.exp(sim_neg / _TEMP)
        lb = -jnp.log(pos / (pos + neg))
        contrib = jnp.where(lane == 16 * li, lb, contrib)

    @pl.when(b == 0)
    def _():
        acc_ref[...] = jnp.zeros((1, 128), jnp.float32)

    acc_ref[...] = acc_ref[...] + contrib

    @pl.when(b == _NB - 1)
    def _():
        s = acc_ref[...]
        lvl_tot = jnp.minimum(jnp.minimum(s[:, 0:1], s[:, 16:17]), s[:, 32:33])
        out_ref[...] = lvl_tot * binv_ref[0, 0]


def _tc_post(iou3, cropT, parts, binv):
    return pl.pallas_call(
        _tc_post_body,
        grid=(_NB,),
        in_specs=[
            pl.BlockSpec((1, _NPI, 1), lambda b: (b, 0, 0)),
            pl.BlockSpec((1, _NLVL, _D), lambda b: (b, 0, 0)),
            pl.BlockSpec((_NPI, 64), lambda b: (b, 0)),
            pl.BlockSpec(memory_space=pltpu.SMEM),
        ],
        out_specs=pl.BlockSpec((1, 1), lambda b: (0, 0)),
        out_shape=jax.ShapeDtypeStruct((1, 1), jnp.float32),
        scratch_shapes=[pltpu.VMEM((1, 128), jnp.float32)],
    )(iou3, cropT, parts, binv)


def kernel(box_cls_feat_con, crop_feat_con, batch_size, ious):
    cropT = jnp.transpose(crop_feat_con, (1, 0, 2))
    binv = (1.0 / jnp.asarray(batch_size, jnp.float32)).reshape(1, 1)
    iou3 = ious.reshape(_NB, _NPI, 1)
    parts = _stage1_sc(box_cls_feat_con, cropT)
    loss = _tc_post(iou3, cropT, parts, binv)
    return loss[0, 0]


# speedup vs baseline: 1.1755x; 1.1755x over previous
import functools

import jax
import jax.numpy as jnp
from jax import lax
from jax.experimental import pallas as pl
from jax.experimental.pallas import tpu as pltpu
from jax.experimental.pallas import tpu_sc as plsc

_TEMP = 0.2
_THRES = 0.4
_NPI = 256
_D = 512
_NB = 32
_NLVL = 3


def _stage1_body(iou_ref, crop_ref, box_ref, out_ref):
    x = box_ref[...]
    z = crop_ref[0]
    nb2 = jnp.sum(x * x, axis=1, keepdims=True)
    inv_nb = jax.lax.rsqrt(jnp.maximum(nb2, 1e-24))
    nz2 = jnp.sum(z * z, axis=1, keepdims=True)
    inv_nz = jax.lax.rsqrt(jnp.maximum(nz2, 1e-24))
    zh = z * inv_nz
    dots = jax.lax.dot_general(
        x, zh, (((1,), (1,)), ((), ())),
        preferred_element_type=jnp.float32)
    cos = dots * inv_nb
    mask = (iou_ref[0] >= _THRES).astype(jnp.float32)
    sp = jnp.sum(cos * mask, axis=0, keepdims=True)
    sa = jnp.sum(cos, axis=0, keepdims=True)
    cp = jnp.sum(mask)
    lane = jax.lax.broadcasted_iota(jnp.int32, (1, 128), 1)
    row = jnp.zeros((1, 128), jnp.float32)
    for k in range(_NLVL):
        row = jnp.where(lane == 16 * k, sp[0, k], row)
        row = jnp.where(lane == 16 * (k + 3), sa[0, k], row)
    row = jnp.where(lane == 16 * 6, cp, row)
    out_ref[0] = row


def _stage2_body(stats_ref, binv_ref, out_ref):
    s = stats_ref[:, 0, :]
    cp = s[:, 96:97]
    cn = _NPI - cp
    lvl_tot = None
    for k in range(_NLVL):
        sp = s[:, 16 * k:16 * k + 1]
        sa = s[:, 16 * (k + 3):16 * (k + 3) + 1]
        sn = sa - sp
        sim_pos = -(sp / cp)
        sim_neg = -(sn / cn)
        pos = jnp.exp(sim_pos / _TEMP)
        neg = jnp.exp(sim_neg / _TEMP)
        lb = -jnp.log(pos / (pos + neg))
        lvl = jnp.sum(lb, axis=0, keepdims=True)
        lvl_tot = lvl if lvl_tot is None else jnp.minimum(lvl_tot, lvl)
    out_ref[...] = lvl_tot * binv_ref[0, 0]


def _stage1_tc(box, crop, iou3):
    return pl.pallas_call(
        _stage1_body,
        grid=(_NB,),
        in_specs=[
            pl.BlockSpec((1, _NPI, 1), lambda b: (b, 0, 0)),
            pl.BlockSpec((1, _NLVL, _D), lambda b: (b, 0, 0)),
            pl.BlockSpec((_NPI, _D), lambda b: (b, 0)),
        ],
        out_specs=pl.BlockSpec((1, 1, 128), lambda b: (b, 0, 0)),
        out_shape=jax.ShapeDtypeStruct((_NB, 1, 128), jnp.float32),
    )(iou3, crop, box)


def _stage2(stats, binv):
    return pl.pallas_call(
        _stage2_body,
        in_specs=[
            pl.BlockSpec((_NB, 1, 128), lambda: (0, 0, 0)),
            pl.BlockSpec(memory_space=pltpu.SMEM),
        ],
        out_specs=pl.BlockSpec((1, 1), lambda: (0, 0)),
        out_shape=jax.ShapeDtypeStruct((1, 1), jnp.float32),
    )(stats, binv)


_HALF = 128
_RUNROLL = 8
_NCH = _D // 16


def _rsqrt16(x):
    i = lax.bitcast_convert_type(x, jnp.int32)
    i = 0x5F3759DF - lax.shift_right_logical(i, 1)
    y = lax.bitcast_convert_type(i, jnp.float32)
    for _ in range(3):
        y = y * (1.5 - 0.5 * x * y * y)
    return y


def _bsum16(v):
    return jnp.full((16,), jnp.sum(v), jnp.float32)


_BLK = 64
_NBLK = _NPI // _BLK


def _sc_stage1_body(box_hbm, crop_hbm, out_hbm,
                    xb0, xb1, ob0, ob1, zbuf,
                    sem_a, sem_b, sem_oa, sem_ob):
    wid = lax.axis_index("s") * 2 + lax.axis_index("c")
    base_row = wid * _NPI
    pltpu.sync_copy(crop_hbm.at[wid], zbuf)
    xbufs = [xb0, xb1]
    obufs = [ob0, ob1]
    sems_in = [sem_a, sem_b]
    sems_out = [sem_oa, sem_ob]
    zero = jnp.zeros((16,), jnp.float32)

    in_cp = [None, None]
    out_cp = [None, None]
    in_cp[0] = pltpu.async_copy(
        box_hbm.at[pl.ds(base_row, _BLK)], xb0, sem_a)
    for blk in range(_NBLK):
        par = blk % 2
        in_cp[par].wait()
        if blk + 1 < _NBLK:
            nxt = (blk + 1) % 2
            in_cp[nxt] = pltpu.async_copy(
                box_hbm.at[pl.ds(base_row + (blk + 1) * _BLK, _BLK)],
                xbufs[nxt], sems_in[nxt])
        if out_cp[par] is not None:
            out_cp[par].wait()
        xbuf = xbufs[par]
        obuf = obufs[par]

        def group_body(g, carry, xbuf=xbuf, obuf=obuf):
            r0 = g * _RUNROLL
            accs = [[zero, zero, zero, zero] for _ in range(_RUNROLL)]
            for c in range(_NCH):
                z0 = zbuf[0, pl.ds(16 * c, 16)]
                z1 = zbuf[1, pl.ds(16 * c, 16)]
                z2 = zbuf[2, pl.ds(16 * c, 16)]
                for j in range(_RUNROLL):
                    x = xbuf[r0 + j, pl.ds(16 * c, 16)]
                    a = accs[j]
                    a[0] = a[0] + x * x
                    a[1] = a[1] + x * z0
                    a[2] = a[2] + x * z1
                    a[3] = a[3] + x * z2
            for j in range(_RUNROLL):
                for k in range(4):
                    obuf[r0 + j, pl.ds(16 * k, 16)] = accs[j][k]
            return carry

        lax.fori_loop(0, _BLK // _RUNROLL, group_body, 0)
        out_cp[par] = pltpu.async_copy(
            obuf, out_hbm.at[pl.ds(base_row + blk * _BLK, _BLK)],
            sems_out[par])
    out_cp[0].wait()
    out_cp[1].wait()


def _stage1_sc(box, cropT):
    mesh = plsc.VectorSubcoreMesh(core_axis_name="c", subcore_axis_name="s")
    f = functools.partial(
        pl.kernel,
        out_type=jax.ShapeDtypeStruct((_NB * _NPI, 64), jnp.float32),
        mesh=mesh,
        compiler_params=pltpu.CompilerParams(needs_layout_passes=False),
        scratch_types=[
            pltpu.VMEM((_BLK, _D), jnp.float32),
            pltpu.VMEM((_BLK, _D), jnp.float32),
            pltpu.VMEM((_BLK, 64), jnp.float32),
            pltpu.VMEM((_BLK, 64), jnp.float32),
            pltpu.VMEM((_NLVL, _D), jnp.float32),
            pltpu.SemaphoreType.DMA,
            pltpu.SemaphoreType.DMA,
            pltpu.SemaphoreType.DMA,
            pltpu.SemaphoreType.DMA,
        ],
    )(_sc_stage1_body)
    return f(box, cropT)


def _tc_post_body(iou_ref, crop_ref, parts_ref, binv_ref, out_ref):
    p = parts_ref[...]
    nb2 = jnp.sum(p[:, :, 0:16], axis=2)
    inv_nb = jax.lax.rsqrt(jnp.maximum(nb2, 1e-24))
    z = crop_ref[...]
    nz2 = jnp.sum(z * z, axis=2)
    inv_nz = jax.lax.rsqrt(jnp.maximum(nz2, 1e-24))
    mask = (iou_ref[...] >= _THRES).astype(jnp.float32)
    cp = jnp.sum(mask, axis=1, keepdims=True)
    cn = _NPI - cp
    lvl_tot = None
    for li in range(_NLVL):
        d = jnp.sum(p[:, :, 16 * (li + 1):16 * (li + 2)], axis=2)
        cos = d * inv_nb * inv_nz[:, li:li + 1]
        sp = jnp.sum(cos * mask, axis=1, keepdims=True)
        sa = jnp.sum(cos, axis=1, keepdims=True)
        sim_pos = -(sp / cp)
        sim_neg = -((sa - sp) / cn)
        pos = jnp.exp(sim_pos / _TEMP)
        neg = jnp.exp(sim_neg / _TEMP)
        lb = -jnp.log(pos / (pos + neg))
        lvl = jnp.sum(lb, axis=0, keepdims=True)
        lvl_tot = lvl if lvl_tot is None else jnp.minimum(lvl_tot, lvl)
    out_ref[...] = lvl_tot * binv_ref[0, 0]


def _tc_post(iou2, cropT, parts, binv):
    return pl.pallas_call(
        _tc_post_body,
        in_specs=[
            pl.BlockSpec((_NB, _NPI), lambda: (0, 0)),
            pl.BlockSpec((_NB, _NLVL, _D), lambda: (0, 0, 0)),
            pl.BlockSpec((_NB, _NPI, 64), lambda: (0, 0, 0)),
            pl.BlockSpec(memory_space=pltpu.SMEM),
        ],
        out_specs=pl.BlockSpec((1, 1), lambda: (0, 0)),
        out_shape=jax.ShapeDtypeStruct((1, 1), jnp.float32),
    )(iou2, cropT, parts, binv)


def kernel(box_cls_feat_con, crop_feat_con, batch_size, ious):
    cropT = jnp.transpose(crop_feat_con, (1, 0, 2))
    binv = (1.0 / jnp.asarray(batch_size, jnp.float32)).reshape(1, 1)
    iou2 = ious.reshape(_NB, _NPI)
    parts = _stage1_sc(box_cls_feat_con, cropT)
    loss = _tc_post(iou2, cropT, parts.reshape(_NB, _NPI, 64), binv)
    return loss[0, 0]


# speedup vs baseline: 1.2853x; 1.0934x over previous
import functools

import jax
import jax.numpy as jnp
from jax import lax
from jax.experimental import pallas as pl
from jax.experimental.pallas import tpu as pltpu
from jax.experimental.pallas import tpu_sc as plsc

_TEMP = 0.2
_THRES = 0.4
_NPI = 256
_D = 512
_NB = 32
_NLVL = 3


def _stage1_body(iou_ref, crop_ref, box_ref, out_ref):
    x = box_ref[...]
    z = crop_ref[0]
    nb2 = jnp.sum(x * x, axis=1, keepdims=True)
    inv_nb = jax.lax.rsqrt(jnp.maximum(nb2, 1e-24))
    nz2 = jnp.sum(z * z, axis=1, keepdims=True)
    inv_nz = jax.lax.rsqrt(jnp.maximum(nz2, 1e-24))
    zh = z * inv_nz
    dots = jax.lax.dot_general(
        x, zh, (((1,), (1,)), ((), ())),
        preferred_element_type=jnp.float32)
    cos = dots * inv_nb
    mask = (iou_ref[0] >= _THRES).astype(jnp.float32)
    sp = jnp.sum(cos * mask, axis=0, keepdims=True)
    sa = jnp.sum(cos, axis=0, keepdims=True)
    cp = jnp.sum(mask)
    lane = jax.lax.broadcasted_iota(jnp.int32, (1, 128), 1)
    row = jnp.zeros((1, 128), jnp.float32)
    for k in range(_NLVL):
        row = jnp.where(lane == 16 * k, sp[0, k], row)
        row = jnp.where(lane == 16 * (k + 3), sa[0, k], row)
    row = jnp.where(lane == 16 * 6, cp, row)
    out_ref[0] = row


def _stage2_body(stats_ref, binv_ref, out_ref):
    s = stats_ref[:, 0, :]
    cp = s[:, 96:97]
    cn = _NPI - cp
    lvl_tot = None
    for k in range(_NLVL):
        sp = s[:, 16 * k:16 * k + 1]
        sa = s[:, 16 * (k + 3):16 * (k + 3) + 1]
        sn = sa - sp
        sim_pos = -(sp / cp)
        sim_neg = -(sn / cn)
        pos = jnp.exp(sim_pos / _TEMP)
        neg = jnp.exp(sim_neg / _TEMP)
        lb = -jnp.log(pos / (pos + neg))
        lvl = jnp.sum(lb, axis=0, keepdims=True)
        lvl_tot = lvl if lvl_tot is None else jnp.minimum(lvl_tot, lvl)
    out_ref[...] = lvl_tot * binv_ref[0, 0]


def _stage1_tc(box, crop, iou3):
    return pl.pallas_call(
        _stage1_body,
        grid=(_NB,),
        in_specs=[
            pl.BlockSpec((1, _NPI, 1), lambda b: (b, 0, 0)),
            pl.BlockSpec((1, _NLVL, _D), lambda b: (b, 0, 0)),
            pl.BlockSpec((_NPI, _D), lambda b: (b, 0)),
        ],
        out_specs=pl.BlockSpec((1, 1, 128), lambda b: (b, 0, 0)),
        out_shape=jax.ShapeDtypeStruct((_NB, 1, 128), jnp.float32),
    )(iou3, crop, box)


def _stage2(stats, binv):
    return pl.pallas_call(
        _stage2_body,
        in_specs=[
            pl.BlockSpec((_NB, 1, 128), lambda: (0, 0, 0)),
            pl.BlockSpec(memory_space=pltpu.SMEM),
        ],
        out_specs=pl.BlockSpec((1, 1), lambda: (0, 0)),
        out_shape=jax.ShapeDtypeStruct((1, 1), jnp.float32),
    )(stats, binv)


_HALF = 128
_RUNROLL = 8
_NCH = _D // 16


def _rsqrt16(x):
    i = lax.bitcast_convert_type(x, jnp.int32)
    i = 0x5F3759DF - lax.shift_right_logical(i, 1)
    y = lax.bitcast_convert_type(i, jnp.float32)
    for _ in range(3):
        y = y * (1.5 - 0.5 * x * y * y)
    return y


def _bsum16(v):
    return jnp.full((16,), jnp.sum(v), jnp.float32)


_BLK = 64
_NBLK = _NPI // _BLK


def _sc_stage1_body(box_hbm, crop_hbm, out_hbm,
                    xb0, xb1, ob0, ob1, zbuf,
                    sem_a, sem_b, sem_oa, sem_ob):
    wid = lax.axis_index("s") * 2 + lax.axis_index("c")
    base_row = wid * _NPI
    pltpu.sync_copy(crop_hbm.at[wid], zbuf)
    xbufs = [xb0, xb1]
    obufs = [ob0, ob1]
    sems_in = [sem_a, sem_b]
    sems_out = [sem_oa, sem_ob]
    zero = jnp.zeros((16,), jnp.float32)

    in_cp = [None, None]
    out_cp = [None, None]
    in_cp[0] = pltpu.async_copy(
        box_hbm.at[pl.ds(base_row, _BLK)], xb0, sem_a)
    for blk in range(_NBLK):
        par = blk % 2
        in_cp[par].wait()
        if blk + 1 < _NBLK:
            nxt = (blk + 1) % 2
            in_cp[nxt] = pltpu.async_copy(
                box_hbm.at[pl.ds(base_row + (blk + 1) * _BLK, _BLK)],
                xbufs[nxt], sems_in[nxt])
        if out_cp[par] is not None:
            out_cp[par].wait()
        xbuf = xbufs[par]
        obuf = obufs[par]

        def group_body(g, carry, xbuf=xbuf, obuf=obuf):
            r0 = g * _RUNROLL
            accs = [[zero, zero, zero, zero] for _ in range(_RUNROLL)]
            for c in range(_NCH):
                z0 = zbuf[0, pl.ds(16 * c, 16)]
                z1 = zbuf[1, pl.ds(16 * c, 16)]
                z2 = zbuf[2, pl.ds(16 * c, 16)]
                for j in range(_RUNROLL):
                    x = xbuf[r0 + j, pl.ds(16 * c, 16)]
                    a = accs[j]
                    a[0] = a[0] + x * x
                    a[1] = a[1] + x * z0
                    a[2] = a[2] + x * z1
                    a[3] = a[3] + x * z2
            for j in range(_RUNROLL):
                for k in range(4):
                    obuf[r0 + j, pl.ds(16 * k, 16)] = accs[j][k]
            return carry

        lax.fori_loop(0, _BLK // _RUNROLL, group_body, 0)
        out_cp[par] = pltpu.async_copy(
            obuf, out_hbm.at[pl.ds(base_row + blk * _BLK, _BLK)],
            sems_out[par])
    out_cp[0].wait()
    out_cp[1].wait()


def _stage1_sc(box, cropT):
    mesh = plsc.VectorSubcoreMesh(core_axis_name="c", subcore_axis_name="s")
    f = functools.partial(
        pl.kernel,
        out_type=jax.ShapeDtypeStruct((_NB * _NPI, 64), jnp.float32),
        mesh=mesh,
        compiler_params=pltpu.CompilerParams(needs_layout_passes=False),
        scratch_types=[
            pltpu.VMEM((_BLK, _D), jnp.float32),
            pltpu.VMEM((_BLK, _D), jnp.float32),
            pltpu.VMEM((_BLK, 64), jnp.float32),
            pltpu.VMEM((_BLK, 64), jnp.float32),
            pltpu.VMEM((_NLVL, _D), jnp.float32),
            pltpu.SemaphoreType.DMA,
            pltpu.SemaphoreType.DMA,
            pltpu.SemaphoreType.DMA,
            pltpu.SemaphoreType.DMA,
        ],
    )(_sc_stage1_body)
    return f(box, cropT)


def _tc_post_body(iou_ref, crop_ref, parts_ref, binv_ref, out_ref):
    p2 = parts_ref[...]
    subl = jax.lax.broadcasted_iota(jnp.int32, (64, 128), 0)
    lane = jax.lax.broadcasted_iota(jnp.int32, (64, 128), 1)
    sel = (subl // 16 == lane).astype(jnp.float32)
    r = jax.lax.dot_general(
        p2, sel, (((1,), (0,)), ((), ())),
        preferred_element_type=jnp.float32)
    r3 = r.reshape(_NB, _NPI, 128)
    nb2 = r3[:, :, 0:1]
    inv_nb = jax.lax.rsqrt(jnp.maximum(nb2, 1e-24))
    z = crop_ref[...]
    nz2 = jnp.sum(z * z, axis=2, keepdims=True)
    inv_nz = jax.lax.rsqrt(jnp.maximum(nz2, 1e-24))
    mask = (iou_ref[...] >= _THRES).astype(jnp.float32)
    cp = jnp.sum(mask, axis=1, keepdims=True)
    cn = _NPI - cp
    lvl_tot = None
    for li in range(_NLVL):
        d = r3[:, :, li + 1:li + 2]
        cos = d * inv_nb * inv_nz[:, li:li + 1, :]
        sp = jnp.sum(cos * mask, axis=1, keepdims=True)
        sa = jnp.sum(cos, axis=1, keepdims=True)
        sim_pos = -(sp / cp)
        sim_neg = -((sa - sp) / cn)
        pos = jnp.exp(sim_pos / _TEMP)
        neg = jnp.exp(sim_neg / _TEMP)
        lb = -jnp.log(pos / (pos + neg))
        lvl = jnp.sum(lb, axis=0, keepdims=True)
        lvl_tot = lvl if lvl_tot is None else jnp.minimum(lvl_tot, lvl)
    out_ref[...] = lvl_tot[0] * binv_ref[0, 0]


def _tc_post(iou3, cropT, parts, binv):
    return pl.pallas_call(
        _tc_post_body,
        in_specs=[
            pl.BlockSpec((_NB, _NPI, 1), lambda: (0, 0, 0)),
            pl.BlockSpec((_NB, _NLVL, _D), lambda: (0, 0, 0)),
            pl.BlockSpec((_NB * _NPI, 64), lambda: (0, 0)),
            pl.BlockSpec(memory_space=pltpu.SMEM),
        ],
        out_specs=pl.BlockSpec((1, 1), lambda: (0, 0)),
        out_shape=jax.ShapeDtypeStruct((1, 1), jnp.float32),
    )(iou3, cropT, parts, binv)


def kernel(box_cls_feat_con, crop_feat_con, batch_size, ious):
    cropT = jnp.transpose(crop_feat_con, (1, 0, 2))
    binv = (1.0 / jnp.asarray(batch_size, jnp.float32)).reshape(1, 1)
    iou3 = ious.reshape(_NB, _NPI, 1)
    parts = _stage1_sc(box_cls_feat_con, cropT)
    loss = _tc_post(iou3, cropT, parts, binv)
    return loss[0, 0]


# speedup vs baseline: 1.3679x; 1.0643x over previous
import functools

import jax
import jax.numpy as jnp
from jax import lax
from jax.experimental import pallas as pl
from jax.experimental.pallas import tpu as pltpu
from jax.experimental.pallas import tpu_sc as plsc

_TEMP = 0.2
_THRES = 0.4
_NPI = 256
_D = 512
_NB = 32
_NLVL = 3


def _stage1_body(iou_ref, crop_ref, box_ref, out_ref):
    x = box_ref[...]
    z = crop_ref[0]
    nb2 = jnp.sum(x * x, axis=1, keepdims=True)
    inv_nb = jax.lax.rsqrt(jnp.maximum(nb2, 1e-24))
    nz2 = jnp.sum(z * z, axis=1, keepdims=True)
    inv_nz = jax.lax.rsqrt(jnp.maximum(nz2, 1e-24))
    zh = z * inv_nz
    dots = jax.lax.dot_general(
        x, zh, (((1,), (1,)), ((), ())),
        preferred_element_type=jnp.float32)
    cos = dots * inv_nb
    mask = (iou_ref[0] >= _THRES).astype(jnp.float32)
    sp = jnp.sum(cos * mask, axis=0, keepdims=True)
    sa = jnp.sum(cos, axis=0, keepdims=True)
    cp = jnp.sum(mask)
    lane = jax.lax.broadcasted_iota(jnp.int32, (1, 128), 1)
    row = jnp.zeros((1, 128), jnp.float32)
    for k in range(_NLVL):
        row = jnp.where(lane == 16 * k, sp[0, k], row)
        row = jnp.where(lane == 16 * (k + 3), sa[0, k], row)
    row = jnp.where(lane == 16 * 6, cp, row)
    out_ref[0] = row


def _stage2_body(stats_ref, binv_ref, out_ref):
    s = stats_ref[:, 0, :]
    cp = s[:, 96:97]
    cn = _NPI - cp
    lvl_tot = None
    for k in range(_NLVL):
        sp = s[:, 16 * k:16 * k + 1]
        sa = s[:, 16 * (k + 3):16 * (k + 3) + 1]
        sn = sa - sp
        sim_pos = -(sp / cp)
        sim_neg = -(sn / cn)
        pos = jnp.exp(sim_pos / _TEMP)
        neg = jnp.exp(sim_neg / _TEMP)
        lb = -jnp.log(pos / (pos + neg))
        lvl = jnp.sum(lb, axis=0, keepdims=True)
        lvl_tot = lvl if lvl_tot is None else jnp.minimum(lvl_tot, lvl)
    out_ref[...] = lvl_tot * binv_ref[0, 0]


def _stage1_tc(box, crop, iou3):
    return pl.pallas_call(
        _stage1_body,
        grid=(_NB,),
        in_specs=[
            pl.BlockSpec((1, _NPI, 1), lambda b: (b, 0, 0)),
            pl.BlockSpec((1, _NLVL, _D), lambda b: (b, 0, 0)),
            pl.BlockSpec((_NPI, _D), lambda b: (b, 0)),
        ],
        out_specs=pl.BlockSpec((1, 1, 128), lambda b: (b, 0, 0)),
        out_shape=jax.ShapeDtypeStruct((_NB, 1, 128), jnp.float32),
    )(iou3, crop, box)


def _stage2(stats, binv):
    return pl.pallas_call(
        _stage2_body,
        in_specs=[
            pl.BlockSpec((_NB, 1, 128), lambda: (0, 0, 0)),
            pl.BlockSpec(memory_space=pltpu.SMEM),
        ],
        out_specs=pl.BlockSpec((1, 1), lambda: (0, 0)),
        out_shape=jax.ShapeDtypeStruct((1, 1), jnp.float32),
    )(stats, binv)


_HALF = 128
_RUNROLL = 8
_NCH = _D // 16


def _rsqrt16(x):
    i = lax.bitcast_convert_type(x, jnp.int32)
    i = 0x5F3759DF - lax.shift_right_logical(i, 1)
    y = lax.bitcast_convert_type(i, jnp.float32)
    for _ in range(3):
        y = y * (1.5 - 0.5 * x * y * y)
    return y


def _bsum16(v):
    return jnp.full((16,), jnp.sum(v), jnp.float32)


_BLK = 64
_NBLK = _NPI // _BLK


def _sc_stage1_body(box_hbm, crop_hbm, out_hbm,
                    xb0, xb1, ob0, ob1, zbuf,
                    sem_a, sem_b, sem_oa, sem_ob):
    wid = lax.axis_index("s") * 2 + lax.axis_index("c")
    base_row = wid * _NPI
    pltpu.sync_copy(crop_hbm.at[wid], zbuf)
    xbufs = [xb0, xb1]
    obufs = [ob0, ob1]
    sems_in = [sem_a, sem_b]
    sems_out = [sem_oa, sem_ob]
    zero = jnp.zeros((16,), jnp.float32)

    in_cp = [None, None]
    out_cp = [None, None]
    in_cp[0] = pltpu.async_copy(
        box_hbm.at[pl.ds(base_row, _BLK)], xb0, sem_a)
    for blk in range(_NBLK):
        par = blk % 2
        in_cp[par].wait()
        if blk + 1 < _NBLK:
            nxt = (blk + 1) % 2
            in_cp[nxt] = pltpu.async_copy(
                box_hbm.at[pl.ds(base_row + (blk + 1) * _BLK, _BLK)],
                xbufs[nxt], sems_in[nxt])
        if out_cp[par] is not None:
            out_cp[par].wait()
        xbuf = xbufs[par]
        obuf = obufs[par]

        def group_body(g, carry, xbuf=xbuf, obuf=obuf):
            r0 = g * _RUNROLL
            accs = [[zero, zero, zero] for _ in range(_RUNROLL)]
            for c in range(_NCH):
                z0 = zbuf[0, pl.ds(16 * c, 16)]
                z1 = zbuf[1, pl.ds(16 * c, 16)]
                z2 = zbuf[2, pl.ds(16 * c, 16)]
                for j in range(_RUNROLL):
                    x = xbuf[r0 + j, pl.ds(16 * c, 16)]
                    a = accs[j]
                    a[0] = a[0] + x * z0
                    a[1] = a[1] + x * z1
                    a[2] = a[2] + x * z2
            for j in range(_RUNROLL):
                for k in range(3):
                    obuf[r0 + j, pl.ds(16 * k, 16)] = accs[j][k]
            return carry

        lax.fori_loop(0, _BLK // _RUNROLL, group_body, 0)
        out_cp[par] = pltpu.async_copy(
            obuf, out_hbm.at[pl.ds(base_row + blk * _BLK, _BLK)],
            sems_out[par])
    out_cp[0].wait()
    out_cp[1].wait()


def _stage1_sc(box, cropT):
    mesh = plsc.VectorSubcoreMesh(core_axis_name="c", subcore_axis_name="s")
    f = functools.partial(
        pl.kernel,
        out_type=jax.ShapeDtypeStruct((_NB * _NPI, 48), jnp.float32),
        mesh=mesh,
        compiler_params=pltpu.CompilerParams(needs_layout_passes=False),
        scratch_types=[
            pltpu.VMEM((_BLK, _D), jnp.float32),
            pltpu.VMEM((_BLK, _D), jnp.float32),
            pltpu.VMEM((_BLK, 48), jnp.float32),
            pltpu.VMEM((_BLK, 48), jnp.float32),
            pltpu.VMEM((_NLVL, _D), jnp.float32),
            pltpu.SemaphoreType.DMA,
            pltpu.SemaphoreType.DMA,
            pltpu.SemaphoreType.DMA,
            pltpu.SemaphoreType.DMA,
        ],
    )(_sc_stage1_body)
    return f(box, cropT)


def _tc_post_body(iou_ref, crop_ref, parts_ref, nb2_ref, binv_ref, out_ref):
    p2 = parts_ref[...]
    subl = jax.lax.broadcasted_iota(jnp.int32, (48, 128), 0)
    lane = jax.lax.broadcasted_iota(jnp.int32, (48, 128), 1)
    sel = (subl // 16 == lane).astype(jnp.float32)
    r = jax.lax.dot_general(
        p2, sel, (((1,), (0,)), ((), ())),
        preferred_element_type=jnp.float32)
    r3 = r.reshape(_NB, _NPI, 128)
    nb2 = nb2_ref[...]
    inv_nb = jax.lax.rsqrt(jnp.maximum(nb2, 1e-24))
    z = crop_ref[...]
    nz2 = jnp.sum(z * z, axis=2, keepdims=True)
    inv_nz = jax.lax.rsqrt(jnp.maximum(nz2, 1e-24))
    mask = (iou_ref[...] >= _THRES).astype(jnp.float32)
    cp = jnp.sum(mask, axis=1, keepdims=True)
    cn = _NPI - cp
    lvl_tot = None
    for li in range(_NLVL):
        d = r3[:, :, li:li + 1]
        cos = d * inv_nb * inv_nz[:, li:li + 1, :]
        sp = jnp.sum(cos * mask, axis=1, keepdims=True)
        sa = jnp.sum(cos, axis=1, keepdims=True)
        sim_pos = -(sp / cp)
        sim_neg = -((sa - sp) / cn)
        pos = jnp.exp(sim_pos / _TEMP)
        neg = jnp.exp(sim_neg / _TEMP)
        lb = -jnp.log(pos / (pos + neg))
        lvl = jnp.sum(lb, axis=0, keepdims=True)
        lvl_tot = lvl if lvl_tot is None else jnp.minimum(lvl_tot, lvl)
    out_ref[...] = lvl_tot[0] * binv_ref[0, 0]


def _tc_post(iou3, cropT, parts, nb2, binv):
    return pl.pallas_call(
        _tc_post_body,
        in_specs=[
            pl.BlockSpec((_NB, _NPI, 1), lambda: (0, 0, 0)),
            pl.BlockSpec((_NB, _NLVL, _D), lambda: (0, 0, 0)),
            pl.BlockSpec((_NB * _NPI, 48), lambda: (0, 0)),
            pl.BlockSpec((_NB, _NPI, 1), lambda: (0, 0, 0)),
            pl.BlockSpec(memory_space=pltpu.SMEM),
        ],
        out_specs=pl.BlockSpec((1, 1), lambda: (0, 0)),
        out_shape=jax.ShapeDtypeStruct((1, 1), jnp.float32),
    )(iou3, cropT, parts, nb2, binv)


def _tc_norms_body(box_ref, out_ref):
    x = box_ref[...]
    out_ref[...] = jnp.sum(x * x, axis=1, keepdims=True)


def _tc_norms(box):
    return pl.pallas_call(
        _tc_norms_body,
        grid=(4,),
        in_specs=[pl.BlockSpec((_NB * _NPI // 4, _D), lambda b: (b, 0))],
        out_specs=pl.BlockSpec((_NB * _NPI // 4, 1), lambda b: (b, 0)),
        out_shape=jax.ShapeDtypeStruct((_NB * _NPI, 1), jnp.float32),
    )(box)


def kernel(box_cls_feat_con, crop_feat_con, batch_size, ious):
    cropT = jnp.transpose(crop_feat_con, (1, 0, 2))
    binv = (1.0 / jnp.asarray(batch_size, jnp.float32)).reshape(1, 1)
    iou3 = ious.reshape(_NB, _NPI, 1)
    parts = _stage1_sc(box_cls_feat_con, cropT)
    nb2 = _tc_norms(box_cls_feat_con).reshape(_NB, _NPI, 1)
    loss = _tc_post(iou3, cropT, parts, nb2, binv)
    return loss[0, 0]


# speedup vs baseline: 1.3682x; 1.0002x over previous
import functools

import jax
import jax.numpy as jnp
from jax import lax
from jax.experimental import pallas as pl
from jax.experimental.pallas import tpu as pltpu
from jax.experimental.pallas import tpu_sc as plsc

_TEMP = 0.2
_THRES = 0.4
_NPI = 256
_D = 512
_NB = 32
_NLVL = 3

_RUNROLL = 8
_NCH = _D // 16
_BLK = 64
_NBLK = _NPI // _BLK


def _sc_stage1_body(box_hbm, crop_hbm, out_hbm,
                    xb0, xb1, ob0, ob1, zbuf,
                    sem_a, sem_b, sem_oa, sem_ob):
    wid = lax.axis_index("s") * 2 + lax.axis_index("c")
    base_row = wid * _NPI
    pltpu.sync_copy(crop_hbm.at[wid], zbuf)
    xbufs = [xb0, xb1]
    obufs = [ob0, ob1]
    sems_in = [sem_a, sem_b]
    sems_out = [sem_oa, sem_ob]
    zero = jnp.zeros((16,), jnp.float32)

    in_cp = [None, None]
    out_cp = [None, None]
    in_cp[0] = pltpu.async_copy(
        box_hbm.at[pl.ds(base_row, _BLK)], xb0, sem_a)
    for blk in range(_NBLK):
        par = blk % 2
        in_cp[par].wait()
        if blk + 1 < _NBLK:
            nxt = (blk + 1) % 2
            in_cp[nxt] = pltpu.async_copy(
                box_hbm.at[pl.ds(base_row + (blk + 1) * _BLK, _BLK)],
                xbufs[nxt], sems_in[nxt])
        if out_cp[par] is not None:
            out_cp[par].wait()
        xbuf = xbufs[par]
        obuf = obufs[par]

        def group_body(g, carry, xbuf=xbuf, obuf=obuf):
            r0 = g * _RUNROLL
            accs = [[zero, zero, zero] for _ in range(_RUNROLL)]
            for c in range(_NCH):
                z0 = zbuf[0, pl.ds(16 * c, 16)]
                z1 = zbuf[1, pl.ds(16 * c, 16)]
                z2 = zbuf[2, pl.ds(16 * c, 16)]
                for j in range(_RUNROLL):
                    x = xbuf[r0 + j, pl.ds(16 * c, 16)]
                    a = accs[j]
                    a[0] = a[0] + x * z0
                    a[1] = a[1] + x * z1
                    a[2] = a[2] + x * z2
            for j in range(_RUNROLL):
                for k in range(_NLVL):
                    obuf[r0 + j, pl.ds(16 * k, 16)] = accs[j][k]
            return carry

        lax.fori_loop(0, _BLK // _RUNROLL, group_body, 0)
        out_cp[par] = pltpu.async_copy(
            obuf, out_hbm.at[pl.ds(base_row + blk * _BLK, _BLK)],
            sems_out[par])
    out_cp[0].wait()
    out_cp[1].wait()


def _stage1_sc(box, cropT):
    mesh = plsc.VectorSubcoreMesh(core_axis_name="c", subcore_axis_name="s")
    f = functools.partial(
        pl.kernel,
        out_type=jax.ShapeDtypeStruct((_NB * _NPI, 48), jnp.float32),
        mesh=mesh,
        compiler_params=pltpu.CompilerParams(needs_layout_passes=False),
        scratch_types=[
            pltpu.VMEM((_BLK, _D), jnp.float32),
            pltpu.VMEM((_BLK, _D), jnp.float32),
            pltpu.VMEM((_BLK, 48), jnp.float32),
            pltpu.VMEM((_BLK, 48), jnp.float32),
            pltpu.VMEM((_NLVL, _D), jnp.float32),
            pltpu.SemaphoreType.DMA,
            pltpu.SemaphoreType.DMA,
            pltpu.SemaphoreType.DMA,
            pltpu.SemaphoreType.DMA,
        ],
    )(_sc_stage1_body)
    return f(box, cropT)


def _tc_norms_body(box_ref, out_ref):
    x = box_ref[...]
    out_ref[...] = jnp.sum(x * x, axis=1, keepdims=True)


def _tc_norms(box):
    return pl.pallas_call(
        _tc_norms_body,
        grid=(4,),
        in_specs=[pl.BlockSpec((_NB * _NPI // 4, _D), lambda b: (b, 0))],
        out_specs=pl.BlockSpec((_NB * _NPI // 4, 1), lambda b: (b, 0)),
        out_shape=jax.ShapeDtypeStruct((_NB * _NPI, 1), jnp.float32),
    )(box)


def _tc_post_body(iou_ref, crop_ref, parts_ref, nb2_ref, binv_ref, out_ref):
    p2 = parts_ref[...]
    subl = jax.lax.broadcasted_iota(jnp.int32, (48, 128), 0)
    lane = jax.lax.broadcasted_iota(jnp.int32, (48, 128), 1)
    sel = (subl // 16 == lane).astype(jnp.float32)
    r = jax.lax.dot_general(
        p2, sel, (((1,), (0,)), ((), ())),
        preferred_element_type=jnp.float32)
    r3 = r.reshape(_NB, _NPI, 128)
    nb2 = nb2_ref[...]
    inv_nb = jax.lax.rsqrt(jnp.maximum(nb2, 1e-24))
    z = crop_ref[...]
    nz2 = jnp.sum(z * z, axis=2, keepdims=True)
    inv_nz = jax.lax.rsqrt(jnp.maximum(nz2, 1e-24))
    mask = (iou_ref[...] >= _THRES).astype(jnp.float32)
    cp = jnp.sum(mask, axis=1, keepdims=True)
    cn = _NPI - cp
    lvl_tot = None
    for li in range(_NLVL):
        d = r3[:, :, li:li + 1]
        cos = d * inv_nb * inv_nz[:, li:li + 1, :]
        sp = jnp.sum(cos * mask, axis=1, keepdims=True)
        sa = jnp.sum(cos, axis=1, keepdims=True)
        sim_pos = -(sp / cp)
        sim_neg = -((sa - sp) / cn)
        pos = jnp.exp(sim_pos / _TEMP)
        neg = jnp.exp(sim_neg / _TEMP)
        lb = -jnp.log(pos / (pos + neg))
        lvl = jnp.sum(lb, axis=0, keepdims=True)
        lvl_tot = lvl if lvl_tot is None else jnp.minimum(lvl_tot, lvl)
    out_ref[...] = lvl_tot[0] * binv_ref[0, 0]


def _tc_post(iou3, cropT, parts, nb2, binv):
    return pl.pallas_call(
        _tc_post_body,
        in_specs=[
            pl.BlockSpec((_NB, _NPI, 1), lambda: (0, 0, 0)),
            pl.BlockSpec((_NB, _NLVL, _D), lambda: (0, 0, 0)),
            pl.BlockSpec((_NB * _NPI, 48), lambda: (0, 0)),
            pl.BlockSpec((_NB, _NPI, 1), lambda: (0, 0, 0)),
            pl.BlockSpec(memory_space=pltpu.SMEM),
        ],
        out_specs=pl.BlockSpec((1, 1), lambda: (0, 0)),
        out_shape=jax.ShapeDtypeStruct((1, 1), jnp.float32),
    )(iou3, cropT, parts, nb2, binv)


def kernel(box_cls_feat_con, crop_feat_con, batch_size, ious):
    cropT = jnp.transpose(crop_feat_con, (1, 0, 2))
    binv = (1.0 / jnp.asarray(batch_size, jnp.float32)).reshape(1, 1)
    iou3 = ious.reshape(_NB, _NPI, 1)
    nb2 = _tc_norms(box_cls_feat_con).reshape(_NB, _NPI, 1)
    parts = _stage1_sc(box_cls_feat_con, cropT)
    loss = _tc_post(iou3, cropT, parts, nb2, binv)
    return loss[0, 0]
